# baseline (speedup 1.0000x reference)
import functools

import jax
import jax.numpy as jnp
from jax import lax
from jax.experimental import pallas as pl
from jax.experimental.pallas import tpu as pltpu
from jax.experimental.pallas import tpu_sc as plsc

D_MODEL = 1024
NUM_EMB = 4

_NC = 2
_NS = 16
_NW = _NC * _NS

_TOTAL = 4 * 8192
_BPW = _TOTAL // _NW
_R = 32
_NCH = _BPW // _R
_NG = _R // 16


def _sc_body(ids_hbm, table_hbm, out_hbm, idx_v, table_v, bufA, bufB,
             ssemA, ssemB):
    sid = lax.axis_index("s")
    wid = sid * _NC + lax.axis_index("c")
    base = wid * _BPW

    pltpu.sync_copy(table_hbm, table_v)
    pltpu.sync_copy(ids_hbm.at[pl.ds(base, _BPW)], idx_v)

    iota = lax.iota(jnp.int32, 16)
    dst = [(iota + g * 16) * D_MODEL for g in range(_NG)]

    ssems = (ssemA, ssemB)
    sd = [None] * _NCH

    for i in range(_NCH):
        if i >= 2:
            sd[i - 2].wait()
        buf = bufA if i % 2 == 0 else bufB
        src = [idx_v[pl.ds(i * _R + g * 16, 16)] * D_MODEL for g in range(_NG)]

        @plsc.parallel_loop(0, D_MODEL, unroll=4)
        def _(j):
            for g in range(_NG):
                vals = plsc.load_gather(table_v, [src[g] + j])
                plsc.store_scatter(buf, [dst[g] + j], vals)

        sd[i] = pltpu.async_copy(
            buf, out_hbm.at[pl.ds((base + i * _R) * D_MODEL, _R * D_MODEL)],
            ssems[i % 2])
    sd[_NCH - 2].wait()
    sd[_NCH - 1].wait()


@jax.jit
def _sc_lookup(ids_flat, table_flat):
    mesh = plsc.VectorSubcoreMesh(
        core_axis_name="c", subcore_axis_name="s",
        num_cores=_NC, num_subcores=_NS)
    f = functools.partial(
        pl.kernel,
        out_type=jax.ShapeDtypeStruct((_TOTAL * D_MODEL,), jnp.float32),
        mesh=mesh,
        scratch_types=[
            pltpu.VMEM((_BPW,), jnp.int32),
            pltpu.VMEM((NUM_EMB * D_MODEL,), jnp.float32),
            pltpu.VMEM((_R * D_MODEL,), jnp.float32),
            pltpu.VMEM((_R * D_MODEL,), jnp.float32),
            pltpu.SemaphoreType.DMA,
            pltpu.SemaphoreType.DMA,
        ],
        compiler_params=pltpu.CompilerParams(needs_layout_passes=False),
    )(_sc_body)
    return f(ids_flat, table_flat)


def kernel(postion_ids, table):
    B, S = postion_ids.shape
    ids_flat = postion_ids.reshape(B * S).astype(jnp.int32)
    out = _sc_lookup(ids_flat, table.reshape(NUM_EMB * D_MODEL))
    return out.reshape(B, S, D_MODEL)


# speedup vs baseline: 1.4727x; 1.4727x over previous
import functools

import jax
import jax.numpy as jnp
from jax import lax
from jax.experimental import pallas as pl
from jax.experimental.pallas import tpu as pltpu
from jax.experimental.pallas import tpu_sc as plsc

D_MODEL = 1024
NUM_EMB = 4

_NC = 2
_NS = 16
_NW = _NC * _NS

_TOTAL = 4 * 8192
_BPW = _TOTAL // _NW
_R = 32
_NCH = _BPW // _R
_NG = _R // 16


def _sc_body(ids_hbm, table_hbm, out_hbm, idx_v, table_v, bufA, bufB,
             ssemA, ssemB):
    sid = lax.axis_index("s")
    wid = sid * _NC + lax.axis_index("c")
    base = wid * _BPW

    pltpu.sync_copy(table_hbm, table_v)
    pltpu.sync_copy(ids_hbm.at[pl.ds(base, _BPW)], idx_v)

    iota = lax.iota(jnp.int32, 16)

    ssems = (ssemA, ssemB)
    sd = [None] * _NCH

    for i in range(_NCH):
        if i >= 2:
            sd[i - 2].wait()
        buf = bufA if i % 2 == 0 else bufB

        def _row(r, carry):
            rid = plsc.load_gather(idx_v, [jnp.full((16,), i * _R, jnp.int32) + r])
            src0 = rid * D_MODEL + iota
            rbase = r * D_MODEL
            for j in range(D_MODEL // 16):
                vals = plsc.load_gather(table_v, [src0 + (j * 16)])
                buf[pl.ds(rbase + j * 16, 16)] = vals
            return carry

        lax.fori_loop(0, _R, _row, 0)

        sd[i] = pltpu.async_copy(
            buf, out_hbm.at[pl.ds((base + i * _R) * D_MODEL, _R * D_MODEL)],
            ssems[i % 2])
    sd[_NCH - 2].wait()
    sd[_NCH - 1].wait()


@jax.jit
def _sc_lookup(ids_flat, table_flat):
    mesh = plsc.VectorSubcoreMesh(
        core_axis_name="c", subcore_axis_name="s",
        num_cores=_NC, num_subcores=_NS)
    f = functools.partial(
        pl.kernel,
        out_type=jax.ShapeDtypeStruct((_TOTAL * D_MODEL,), jnp.float32),
        mesh=mesh,
        scratch_types=[
            pltpu.VMEM((_BPW,), jnp.int32),
            pltpu.VMEM((NUM_EMB * D_MODEL,), jnp.float32),
            pltpu.VMEM((_R * D_MODEL,), jnp.float32),
            pltpu.VMEM((_R * D_MODEL,), jnp.float32),
            pltpu.SemaphoreType.DMA,
            pltpu.SemaphoreType.DMA,
        ],
        compiler_params=pltpu.CompilerParams(needs_layout_passes=False),
    )(_sc_body)
    return f(ids_flat, table_flat)


def kernel(postion_ids, table):
    B, S = postion_ids.shape
    ids_flat = postion_ids.reshape(B * S).astype(jnp.int32)
    out = _sc_lookup(ids_flat, table.reshape(NUM_EMB * D_MODEL))
    return out.reshape(B, S, D_MODEL)


# speedup vs baseline: 2.6594x; 1.8058x over previous
import functools

import jax
import jax.numpy as jnp
from jax import lax
from jax.experimental import pallas as pl
from jax.experimental.pallas import tpu as pltpu
from jax.experimental.pallas import tpu_sc as plsc

D_MODEL = 1024
NUM_EMB = 4

_NC = 2
_NS = 16
_NW = _NC * _NS

_TOTAL = 4 * 8192
_BPW = _TOTAL // _NW
_R = 32
_NCH = _BPW // _R
_NG = _R // 16


def _sc_body(ids_hbm, table_hbm, out_hbm, idx_v, table_v, buf, ssemA, ssemB):
    sid = lax.axis_index("s")
    wid = sid * _NC + lax.axis_index("c")
    base = wid * _BPW

    pltpu.sync_copy(table_hbm, table_v)
    pltpu.sync_copy(ids_hbm.at[pl.ds(base, _BPW)], idx_v)

    iota = lax.iota(jnp.int32, 16)
    _RD = _R * D_MODEL

    def scatter_desc(i, par, sem):
        return pltpu.make_async_copy(
            buf.at[pl.ds(par * _RD, _RD)],
            out_hbm.at[pl.ds((base + i * _R) * D_MODEL, _RD)],
            sem)

    def chunk(i, carry):
        par = i % 2
        boff = par * _RD

        @pl.when(i >= 2)
        def _():
            @pl.when(par == 0)
            def _():
                scatter_desc(i - 2, 0, ssemA).wait()

            @pl.when(par == 1)
            def _():
                scatter_desc(i - 2, 1, ssemB).wait()

        @plsc.parallel_loop(0, _R)
        def _(r):
            rid = plsc.load_gather(idx_v, [jnp.full((16,), 0, jnp.int32)
                                           + (i * _R + r)])
            src0 = rid * D_MODEL + iota
            rbase = boff + r * D_MODEL
            for j in range(D_MODEL // 16):
                vals = plsc.load_gather(table_v, [src0 + (j * 16)])
                buf[pl.ds(rbase + j * 16, 16)] = vals

        @pl.when(par == 0)
        def _():
            scatter_desc(i, 0, ssemA).start()

        @pl.when(par == 1)
        def _():
            scatter_desc(i, 1, ssemB).start()

        return carry

    lax.fori_loop(0, _NCH, chunk, 0)
    scatter_desc(_NCH - 2, 0, ssemA).wait()
    scatter_desc(_NCH - 1, 1, ssemB).wait()


@jax.jit
def _sc_lookup(ids_flat, table_flat):
    mesh = plsc.VectorSubcoreMesh(
        core_axis_name="c", subcore_axis_name="s",
        num_cores=_NC, num_subcores=_NS)
    f = functools.partial(
        pl.kernel,
        out_type=jax.ShapeDtypeStruct((_TOTAL * D_MODEL,), jnp.float32),
        mesh=mesh,
        scratch_types=[
            pltpu.VMEM((_BPW,), jnp.int32),
            pltpu.VMEM((NUM_EMB * D_MODEL,), jnp.float32),
            pltpu.VMEM((2 * _R * D_MODEL,), jnp.float32),
            pltpu.SemaphoreType.DMA,
            pltpu.SemaphoreType.DMA,
        ],
        compiler_params=pltpu.CompilerParams(needs_layout_passes=False),
    )(_sc_body)
    return f(ids_flat, table_flat)


def kernel(postion_ids, table):
    B, S = postion_ids.shape
    ids_flat = postion_ids.reshape(B * S).astype(jnp.int32)
    out = _sc_lookup(ids_flat, table.reshape(NUM_EMB * D_MODEL))
    return out.reshape(B, S, D_MODEL)


# speedup vs baseline: 5.0917x; 1.9146x over previous
import functools

import jax
import jax.numpy as jnp
from jax import lax
from jax.experimental import pallas as pl
from jax.experimental.pallas import tpu as pltpu
from jax.experimental.pallas import tpu_sc as plsc

D_MODEL = 1024
NUM_EMB = 4

_NC = 2
_NS = 16
_NW = _NC * _NS

_TOTAL = 4 * 8192
_BPW = _TOTAL // _NW

_GROWS = 512
_RG = 32
_NCHG = _GROWS // _RG

_FROWS = _BPW - _GROWS
_RF = 16
_NCHF = _FROWS // _RF


def _sc_body(ids_hbm, table_hbm, out_hbm, idx_v, table_v, gbuf, fbuf,
             gsemA, gsemB, ssemA, ssemB, fsemA, fsemB):
    sid = lax.axis_index("s")
    wid = sid * _NC + lax.axis_index("c")
    base = wid * _BPW

    pltpu.sync_copy(table_hbm.at[pl.ds(0, NUM_EMB)], table_v)
    pltpu.sync_copy(ids_hbm.at[pl.ds(base, _BPW)], idx_v)
    off = wid * NUM_EMB
    for j in range(_GROWS // 16):
        sl = pl.ds(j * 16, 16)
        idx_v[sl] = idx_v[sl] + off

    iota = lax.iota(jnp.int32, 16)
    gsems = (gsemA, gsemB)
    ssems = (ssemA, ssemB)
    fsems = (fsemA, fsemB)

    def g_gather(g, par):
        return pltpu.make_async_copy(
            table_hbm.at[idx_v.at[pl.ds(g * _RG, _RG)]],
            gbuf.at[par], gsems[par])

    def g_scat(g, par):
        return pltpu.make_async_copy(
            gbuf.at[par], out_hbm.at[pl.ds(base + g * _RG, _RG)], ssems[par])

    def f_scat(i, par):
        return pltpu.make_async_copy(
            fbuf.at[pl.ds(par * _RF, _RF)],
            out_hbm.at[pl.ds(base + _GROWS + i * _RF, _RF)], fsems[par])

    def step(i, carry):
        par_f = i % 2

        @pl.when(i >= 2)
        def _():
            @pl.when(par_f == 0)
            def _():
                f_scat(i - 2, 0).wait()

            @pl.when(par_f == 1)
            def _():
                f_scat(i - 2, 1).wait()

        @pl.when(par_f == 0)
        def _():
            g = i // 2
            par_g = g % 2

            @pl.when(g == 0)
            def _():
                g_gather(0, 0).start()

            @pl.when(par_g == 0)
            def _():
                g_gather(g, 0).wait()
                g_scat(g, 0).start()

            @pl.when(par_g == 1)
            def _():
                g_gather(g, 1).wait()
                g_scat(g, 1).start()

            @pl.when(g >= 1)
            def _():
                @pl.when(par_g == 0)
                def _():
                    g_scat(g - 1, 1).wait()

                @pl.when(par_g == 1)
                def _():
                    g_scat(g - 1, 0).wait()

            @pl.when(g + 1 < _NCHG)
            def _():
                @pl.when(par_g == 0)
                def _():
                    g_gather(g + 1, 1).start()

                @pl.when(par_g == 1)
                def _():
                    g_gather(g + 1, 0).start()

        @plsc.parallel_loop(0, _RF)
        def _(r):
            rid = plsc.load_gather(
                idx_v, [jnp.full((16,), 0, jnp.int32) + (_GROWS + i * _RF + r)])
            row2d = par_f * _RF + r
            for j in range(D_MODEL // 16):
                vals = plsc.load_gather(table_v, [rid, iota + (j * 16)])
                fbuf[row2d, pl.ds(j * 16, 16)] = vals

        @pl.when(par_f == 0)
        def _():
            f_scat(i, 0).start()

        @pl.when(par_f == 1)
        def _():
            f_scat(i, 1).start()

        return carry

    lax.fori_loop(0, _NCHF, step, 0)
    f_scat(_NCHF - 2, 0).wait()
    f_scat(_NCHF - 1, 1).wait()
    g_scat(_NCHG - 1, (_NCHG - 1) % 2).wait()


@jax.jit
def _sc_lookup(ids_flat, table_rep):
    mesh = plsc.VectorSubcoreMesh(
        core_axis_name="c", subcore_axis_name="s",
        num_cores=_NC, num_subcores=_NS)
    f = functools.partial(
        pl.kernel,
        out_type=jax.ShapeDtypeStruct((_TOTAL, D_MODEL), jnp.float32),
        mesh=mesh,
        scratch_types=[
            pltpu.VMEM((_BPW,), jnp.int32),
            pltpu.VMEM((NUM_EMB, D_MODEL), jnp.float32),
            pltpu.VMEM((2, _RG, D_MODEL), jnp.float32),
            pltpu.VMEM((2 * _RF, D_MODEL), jnp.float32),
            pltpu.SemaphoreType.DMA,
            pltpu.SemaphoreType.DMA,
            pltpu.SemaphoreType.DMA,
            pltpu.SemaphoreType.DMA,
            pltpu.SemaphoreType.DMA,
            pltpu.SemaphoreType.DMA,
        ],
        compiler_params=pltpu.CompilerParams(needs_layout_passes=False),
    )(_sc_body)
    return f(ids_flat, table_rep)


def kernel(postion_ids, table):
    B, S = postion_ids.shape
    ids_flat = postion_ids.reshape(B * S).astype(jnp.int32)
    table_rep = jnp.tile(table, (_NW, 1))
    out = _sc_lookup(ids_flat, table_rep)
    return out.reshape(B, S, D_MODEL)
